# Initial kernel scaffold; baseline (speedup 1.0000x reference)
import jax
import jax.numpy as jnp
from jax.experimental import pallas as pl


def kernel(out_states, encoded_context2, encoded_in_domainslots2, context, context_mask, Wg, bg, Wq, Wk, Wpg, bpg):
    raise NotImplementedError("write your pallas kernel here")


# speedup vs baseline: 7.0941x; 7.0941x over previous
import functools

import jax
import jax.numpy as jnp
import numpy as np
from jax.experimental import pallas as pl
from jax.experimental.pallas import tpu as pltpu

_B, _T, _Tc, _D, _V = 2, 256, 1024, 1024, 32000
_VT1 = 3200
_VT2 = 3200


def _attn_body(os_ref, ec_ref, ed_ref, maskf_ref, Wq_ref, Wk_ref, wpg_ref,
               bpg_ref, attn_out, s_out):
    os = os_ref[0]
    ec = ec_ref[0]
    q = jnp.dot(os, Wq_ref[...], preferred_element_type=jnp.float32)
    k = jnp.dot(ec, Wk_ref[...], preferred_element_type=jnp.float32)
    scores = jax.lax.dot_general(q, k, (((1,), (1,)), ((), ())),
                                 preferred_element_type=jnp.float32)
    scores = scores * jnp.float32(1.0 / np.sqrt(_D))
    maskf = maskf_ref[0]
    scores = scores + (1.0 - maskf) * jnp.float32(-1e9)
    m = jnp.max(scores, axis=1, keepdims=True)
    e = jnp.exp(scores - m)
    attn = e / jnp.sum(e, axis=1, keepdims=True)
    cv = jnp.dot(attn, ec, preferred_element_type=jnp.float32)
    ed = ed_ref[0]
    wpg = wpg_ref[...]
    slog = (jnp.dot(os, wpg[0:_D], preferred_element_type=jnp.float32)
            + jnp.dot(cv, wpg[_D:2 * _D], preferred_element_type=jnp.float32)
            + jnp.dot(ed, wpg[2 * _D:3 * _D],
                      preferred_element_type=jnp.float32)
            + bpg_ref[0, 0])
    s = jax.nn.sigmoid(slog)
    attn_out[0] = attn
    s_out[0] = s


def _logit_body(os_ref, Wg_ref, bg_ref, va_out, lse_out, m_acc, s_acc):
    j = pl.program_id(1)

    @pl.when(j == 0)
    def _():
        m_acc[...] = jnp.full((_T, 1), -jnp.inf, jnp.float32)
        s_acc[...] = jnp.zeros((_T, 1), jnp.float32)

    os = os_ref[0]
    va = jax.lax.dot_general(os, Wg_ref[...], (((1,), (1,)), ((), ())),
                             preferred_element_type=jnp.float32)
    va = va + bg_ref[0]
    tm = jnp.max(va, axis=1, keepdims=True)
    new_m = jnp.maximum(m_acc[...], tm)
    s_acc[...] = (s_acc[...] * jnp.exp(m_acc[...] - new_m)
                  + jnp.sum(jnp.exp(va - new_m), axis=1, keepdims=True))
    m_acc[...] = new_m
    va_out[0] = va
    lse_out[0] = m_acc[...] + jnp.log(s_acc[...])


def _combine_body(va_ref, attn_ref, s_ref, lse_ref, ctxT_ref, out_ref):
    j = pl.program_id(1)
    ctx = ctxT_ref[0]
    iota = jax.lax.broadcasted_iota(jnp.int32, (_Tc, _VT2), 1) + j * _VT2
    oh = (ctx == iota).astype(jnp.float32)
    pctx = jnp.dot(attn_ref[0], oh, preferred_element_type=jnp.float32)
    s = s_ref[0]
    lse = lse_ref[0]
    pv = jnp.exp(va_ref[0] - lse)
    out_ref[0] = jnp.log(s * pv + (1.0 - s) * pctx)


def kernel(out_states, encoded_context2, encoded_in_domainslots2, context,
           context_mask, Wg, bg, Wq, Wk, Wpg, bpg):
    maskf = context_mask.astype(jnp.float32).reshape(_B, 1, _Tc)
    ctxT = context.astype(jnp.int32).reshape(_B, _Tc, 1)
    wpg_col = Wpg.reshape(3 * _D, 1)
    bpg2 = bpg.reshape(1, 1)
    bg2 = bg.reshape(1, _V)

    attn, s = pl.pallas_call(
        _attn_body,
        grid=(_B,),
        in_specs=[
            pl.BlockSpec((1, _T, _D), lambda b: (b, 0, 0)),
            pl.BlockSpec((1, _Tc, _D), lambda b: (b, 0, 0)),
            pl.BlockSpec((1, _T, _D), lambda b: (b, 0, 0)),
            pl.BlockSpec((1, 1, _Tc), lambda b: (b, 0, 0)),
            pl.BlockSpec((_D, _D), lambda b: (0, 0)),
            pl.BlockSpec((_D, _D), lambda b: (0, 0)),
            pl.BlockSpec((3 * _D, 1), lambda b: (0, 0)),
            pl.BlockSpec((1, 1), lambda b: (0, 0)),
        ],
        out_specs=[
            pl.BlockSpec((1, _T, _Tc), lambda b: (b, 0, 0)),
            pl.BlockSpec((1, _T, 1), lambda b: (b, 0, 0)),
        ],
        out_shape=[
            jax.ShapeDtypeStruct((_B, _T, _Tc), jnp.float32),
            jax.ShapeDtypeStruct((_B, _T, 1), jnp.float32),
        ],
    )(out_states, encoded_context2, encoded_in_domainslots2, maskf, Wq, Wk,
      wpg_col, bpg2)

    nv1 = _V // _VT1
    va, lse = pl.pallas_call(
        _logit_body,
        grid=(_B, nv1),
        in_specs=[
            pl.BlockSpec((1, _T, _D), lambda b, j: (b, 0, 0)),
            pl.BlockSpec((_VT1, _D), lambda b, j: (j, 0)),
            pl.BlockSpec((1, _VT1), lambda b, j: (0, j)),
        ],
        out_specs=[
            pl.BlockSpec((1, _T, _VT1), lambda b, j: (b, 0, j)),
            pl.BlockSpec((1, _T, 1), lambda b, j: (b, 0, 0)),
        ],
        out_shape=[
            jax.ShapeDtypeStruct((_B, _T, _V), jnp.float32),
            jax.ShapeDtypeStruct((_B, _T, 1), jnp.float32),
        ],
        scratch_shapes=[
            pltpu.VMEM((_T, 1), jnp.float32),
            pltpu.VMEM((_T, 1), jnp.float32),
        ],
        compiler_params=pltpu.CompilerParams(
            dimension_semantics=("arbitrary", "arbitrary")),
    )(out_states, Wg, bg2)

    nv2 = _V // _VT2
    out = pl.pallas_call(
        _combine_body,
        grid=(_B, nv2),
        in_specs=[
            pl.BlockSpec((1, _T, _VT2), lambda b, j: (b, 0, j)),
            pl.BlockSpec((1, _T, _Tc), lambda b, j: (b, 0, 0)),
            pl.BlockSpec((1, _T, 1), lambda b, j: (b, 0, 0)),
            pl.BlockSpec((1, _T, 1), lambda b, j: (b, 0, 0)),
            pl.BlockSpec((1, _Tc, 1), lambda b, j: (b, 0, 0)),
        ],
        out_specs=pl.BlockSpec((1, _T, _VT2), lambda b, j: (b, 0, j)),
        out_shape=jax.ShapeDtypeStruct((_B, _T, _V), jnp.float32),
        compiler_params=pltpu.CompilerParams(
            dimension_semantics=("arbitrary", "arbitrary")),
    )(va, attn, s, lse, ctxT)
    return out
